# Initial kernel scaffold; baseline (speedup 1.0000x reference)
import jax
import jax.numpy as jnp
from jax.experimental import pallas as pl


def kernel(x, edge_index, W1, b1, g1, bt1, W2, b2, g2, bt2, W3, b3, g3, bt3):
    raise NotImplementedError("write your pallas kernel here")


# speedup vs baseline: 7.3041x; 7.3041x over previous
import functools

import jax
import jax.numpy as jnp
from jax import lax
from jax.experimental import pallas as pl
from jax.experimental.pallas import tpu as pltpu
from jax.experimental.pallas import tpu_sc as plsc

NC = 2
NS = 16
NW = NC * NS
L = 16

CHUNK = 128


def _pad_geometry(n, e):
    chunks_t = -(-e // (NW * CHUNK))
    chunks_t = -(-chunks_t // 8) * 8
    e_pad = NW * chunks_t * CHUNK
    rows_t = -(-(n + 1) // NS)
    rows_t = -(-rows_t // 8) * 8
    n_pad = rows_t * NS
    return chunks_t, e_pad, rows_t, n_pad


def _make_deg_kernel(n_pad, chunks_t):
    mesh = plsc.VectorSubcoreMesh(core_axis_name="c", subcore_axis_name="s")

    ept = chunks_t * CHUNK

    @functools.partial(
        pl.kernel,
        out_type=jax.ShapeDtypeStruct((NW, n_pad), jnp.float32),
        mesh=mesh,
        scratch_types=[
            pltpu.VMEM((ept,), jnp.int32),
            pltpu.VMEM((n_pad,), jnp.float32),
        ],
        compiler_params=pltpu.CompilerParams(needs_layout_passes=False),
    )
    def deg_kernel(dst_hbm, out_hbm, dst_v, hist_v):
        cid = lax.axis_index("c")
        sid = lax.axis_index("s")
        wid = cid * NS + sid
        pltpu.sync_copy(dst_hbm.at[pl.ds(wid * ept, ept)], dst_v)

        zeros16 = jnp.zeros((L,), jnp.float32)

        def zbody(i, carry):
            hist_v[pl.ds(i * L, L)] = zeros16
            return carry

        lax.fori_loop(0, n_pad // L, zbody, None)

        ones16 = jnp.ones((L,), jnp.float32)

        def body(i, carry):
            idx = dst_v[pl.ds(i * L, L)]
            plsc.addupdate_scatter(hist_v, [idx], ones16)
            return carry

        lax.fori_loop(0, ept // L, body, None)
        pltpu.sync_copy(hist_v, out_hbm.at[wid])

    return deg_kernel


def _make_agg_kernel(n, n_pad, rows_t, chunks_t):
    mesh = plsc.VectorSubcoreMesh(core_axis_name="c", subcore_axis_name="s")

    @functools.partial(
        pl.kernel,
        out_type=jax.ShapeDtypeStruct((NC, n_pad, 128), jnp.float32),
        mesh=mesh,
        scratch_types=[
            pltpu.VMEM((chunks_t, CHUNK), jnp.int32),
            pltpu.VMEM((chunks_t, CHUNK), jnp.int32),
            pltpu.VMEM((CHUNK, 128), jnp.float32),
            pltpu.VMEM_SHARED((n_pad, 128), jnp.float32),
            pltpu.SemaphoreType.DMA,
        ],
        compiler_params=pltpu.CompilerParams(needs_layout_passes=False),
    )
    def agg_kernel(hp_hbm, src_hbm, dst_hbm, zeros_hbm, out_hbm,
                   src_v, dst_v, rows_v, accum, sem):
        cid = lax.axis_index("c")
        sid = lax.axis_index("s")
        wid = cid * NS + sid
        pltpu.sync_copy(src_hbm.at[pl.ds(wid * chunks_t, chunks_t)], src_v)
        pltpu.sync_copy(dst_hbm.at[pl.ds(wid * chunks_t, chunks_t)], dst_v)
        pltpu.sync_copy(zeros_hbm, accum.at[pl.ds(sid * rows_t, rows_t)])
        plsc.subcore_barrier()

        def body(i, carry):
            pltpu.async_copy(hp_hbm.at[src_v.at[i]], rows_v, sem).wait()
            pltpu.sync_copy(rows_v, accum.at[dst_v.at[i]], add=True)
            return carry

        lax.fori_loop(0, chunks_t, body, None)
        plsc.subcore_barrier()
        pltpu.sync_copy(accum.at[pl.ds(sid * rows_t, rows_t)],
                        out_hbm.at[cid, pl.ds(sid * rows_t, rows_t)])

    return agg_kernel


def _dinv_body(hist_ref, o_ref):
    deg = jnp.sum(hist_ref[...], axis=0, keepdims=True) + 1.0
    o_ref[...] = lax.rsqrt(deg)


def _mm_body(x_ref, w_ref, dinv_ref, o_ref):
    h = jnp.dot(x_ref[...], w_ref[...], preferred_element_type=jnp.float32)
    o_ref[...] = h * dinv_ref[...]


def _epilogue_body(p0_ref, p1_ref, hp_ref, dinv_ref, b_ref, g_ref, bt_ref,
                   o_ref):
    t = (p0_ref[...] + p1_ref[...] + hp_ref[...]) * dinv_ref[...] + b_ref[...]
    mu = jnp.mean(t, axis=0, keepdims=True)
    var = jnp.mean((t - mu) ** 2, axis=0, keepdims=True)
    y = g_ref[...] * (t - mu) * lax.rsqrt(var + 1e-5) + bt_ref[...]
    o_ref[...] = jnp.maximum(y, 0.0)


def kernel(x, edge_index, W1, b1, g1, bt1, W2, b2, g2, bt2, W3, b3, g3, bt3):
    n, d = x.shape
    e = edge_index.shape[1]
    chunks_t, e_pad, rows_t, n_pad = _pad_geometry(n, e)

    src = edge_index[0]
    dst = edge_index[1]
    pad = e_pad - e
    src_p = jnp.concatenate([src, jnp.zeros((pad,), src.dtype)])
    dst_p = jnp.concatenate([dst, jnp.full((pad,), n, dst.dtype)])
    src2d = src_p.reshape(NW * chunks_t, CHUNK)
    dst2d = dst_p.reshape(NW * chunks_t, CHUNK)
    zeros_rows = jnp.zeros((rows_t, d), jnp.float32)

    deg_kernel = _make_deg_kernel(n_pad, chunks_t)
    agg_kernel = _make_agg_kernel(n, n_pad, rows_t, chunks_t)

    hist = deg_kernel(dst_p)

    dinv_row = pl.pallas_call(
        _dinv_body,
        out_shape=jax.ShapeDtypeStruct((1, n_pad), jnp.float32),
    )(hist)
    dinv = dinv_row.reshape(n_pad, 1)[:n]

    mm = pl.pallas_call(
        _mm_body,
        out_shape=jax.ShapeDtypeStruct((n, d), jnp.float32),
    )
    epilogue = pl.pallas_call(
        _epilogue_body,
        out_shape=jax.ShapeDtypeStruct((n, d), jnp.float32),
    )

    h = x
    for (W, b, g, bt) in ((W1, b1, g1, bt1), (W2, b2, g2, bt2),
                          (W3, b3, g3, bt3)):
        hp = mm(h, W, dinv)
        parts = agg_kernel(hp, src2d, dst2d, zeros_rows)
        h = epilogue(parts[0, :n], parts[1, :n], hp, dinv,
                     b.reshape(1, d), g.reshape(1, d), bt.reshape(1, d))
    return h


# speedup vs baseline: 8.3086x; 1.1375x over previous
import functools

import jax
import jax.numpy as jnp
from jax import lax
from jax.experimental import pallas as pl
from jax.experimental.pallas import tpu as pltpu
from jax.experimental.pallas import tpu_sc as plsc

NC = 2
NS = 16
NW = NC * NS
L = 16

CHUNK = 128
NBUF = 2


def _pad_geometry(n, e):
    chunks_t = -(-e // (NW * CHUNK))
    chunks_t = -(-chunks_t // 8) * 8
    e_pad = NW * chunks_t * CHUNK
    rows_t = -(-(n + 1) // NS)
    rows_t = -(-rows_t // 8) * 8
    n_pad = rows_t * NS
    return chunks_t, e_pad, rows_t, n_pad


def _make_deg_kernel(n_pad, chunks_t):
    mesh = plsc.VectorSubcoreMesh(core_axis_name="c", subcore_axis_name="s")

    ept = chunks_t * CHUNK

    @functools.partial(
        pl.kernel,
        out_type=jax.ShapeDtypeStruct((NW, n_pad), jnp.float32),
        mesh=mesh,
        scratch_types=[
            pltpu.VMEM((ept,), jnp.int32),
            pltpu.VMEM((n_pad,), jnp.float32),
        ],
        compiler_params=pltpu.CompilerParams(needs_layout_passes=False),
    )
    def deg_kernel(dst_hbm, out_hbm, dst_v, hist_v):
        cid = lax.axis_index("c")
        sid = lax.axis_index("s")
        wid = cid * NS + sid
        pltpu.sync_copy(dst_hbm.at[pl.ds(wid * ept, ept)], dst_v)

        zeros16 = jnp.zeros((L,), jnp.float32)

        def zbody(i, carry):
            hist_v[pl.ds(i * L, L)] = zeros16
            return carry

        lax.fori_loop(0, n_pad // L, zbody, None)

        ones16 = jnp.ones((L,), jnp.float32)

        def body(i, carry):
            idx = dst_v[pl.ds(i * L, L)]
            plsc.addupdate_scatter(hist_v, [idx], ones16)
            return carry

        lax.fori_loop(0, ept // L, body, None)
        pltpu.sync_copy(hist_v, out_hbm.at[wid])

    return deg_kernel


def _make_agg_kernel(n, n_pad, rows_t, chunks_t):
    mesh = plsc.VectorSubcoreMesh(core_axis_name="c", subcore_axis_name="s")

    blk_sz = chunks_t // 2

    @functools.partial(
        pl.kernel,
        out_type=jax.ShapeDtypeStruct((NC, n_pad, 128), jnp.float32),
        mesh=mesh,
        scratch_types=[
            pltpu.VMEM((blk_sz, CHUNK), jnp.int32),
            pltpu.VMEM((blk_sz, CHUNK), jnp.int32),
            pltpu.VMEM((NBUF, CHUNK, 128), jnp.float32),
            pltpu.VMEM_SHARED((n_pad, 128), jnp.float32),
            pltpu.SemaphoreType.DMA((NBUF,)),
        ],
        compiler_params=pltpu.CompilerParams(needs_layout_passes=False),
    )
    def agg_kernel(hp_hbm, src_hbm, dst_hbm, zeros_hbm, out_hbm,
                   src_v, dst_v, rows_v, accum, sems):
        cid = lax.axis_index("c")
        sid = lax.axis_index("s")
        wid = cid * NS + sid
        pltpu.sync_copy(zeros_hbm, accum.at[pl.ds(sid * rows_t, rows_t)])
        plsc.subcore_barrier()

        groups = blk_sz // NBUF

        for blk in range(2):
            base = wid * chunks_t + blk * blk_sz
            pltpu.sync_copy(src_hbm.at[pl.ds(base, blk_sz)], src_v)
            pltpu.sync_copy(dst_hbm.at[pl.ds(base, blk_sz)], dst_v)

            for k in range(NBUF):
                pltpu.async_copy(hp_hbm.at[src_v.at[k]], rows_v.at[k],
                                 sems.at[k])

            def body(g, carry):
                for k in range(NBUF):
                    i = g * NBUF + k
                    pltpu.make_async_copy(
                        hp_hbm.at[src_v.at[i]], rows_v.at[k],
                        sems.at[k]).wait()
                    pltpu.sync_copy(rows_v.at[k], accum.at[dst_v.at[i]],
                                    add=True)

                    @pl.when(g + 1 < groups)
                    def _():
                        pltpu.async_copy(hp_hbm.at[src_v.at[i + NBUF]],
                                         rows_v.at[k], sems.at[k])

                return carry

            lax.fori_loop(0, groups, body, None)

        plsc.subcore_barrier()
        pltpu.sync_copy(accum.at[pl.ds(sid * rows_t, rows_t)],
                        out_hbm.at[cid, pl.ds(sid * rows_t, rows_t)])

    return agg_kernel


def _dinv_body(hist_ref, o_ref):
    deg = jnp.sum(hist_ref[...], axis=0, keepdims=True) + 1.0
    o_ref[...] = lax.rsqrt(deg)


def _mm_body(x_ref, w_ref, dinv_ref, o_ref):
    h = jnp.dot(x_ref[...], w_ref[...], preferred_element_type=jnp.float32)
    o_ref[...] = h * dinv_ref[...]


def _epilogue_body(p0_ref, p1_ref, hp_ref, dinv_ref, b_ref, g_ref, bt_ref,
                   o_ref):
    t = (p0_ref[...] + p1_ref[...] + hp_ref[...]) * dinv_ref[...] + b_ref[...]
    mu = jnp.mean(t, axis=0, keepdims=True)
    var = jnp.mean((t - mu) ** 2, axis=0, keepdims=True)
    y = g_ref[...] * (t - mu) * lax.rsqrt(var + 1e-5) + bt_ref[...]
    o_ref[...] = jnp.maximum(y, 0.0)


def kernel(x, edge_index, W1, b1, g1, bt1, W2, b2, g2, bt2, W3, b3, g3, bt3):
    n, d = x.shape
    e = edge_index.shape[1]
    chunks_t, e_pad, rows_t, n_pad = _pad_geometry(n, e)

    src = edge_index[0]
    dst = edge_index[1]
    pad = e_pad - e
    src_p = jnp.concatenate([src, jnp.zeros((pad,), src.dtype)])
    dst_p = jnp.concatenate([dst, jnp.full((pad,), n, dst.dtype)])
    src2d = src_p.reshape(NW * chunks_t, CHUNK)
    dst2d = dst_p.reshape(NW * chunks_t, CHUNK)
    zeros_rows = jnp.zeros((rows_t, d), jnp.float32)

    deg_kernel = _make_deg_kernel(n_pad, chunks_t)
    agg_kernel = _make_agg_kernel(n, n_pad, rows_t, chunks_t)

    hist = deg_kernel(dst_p)

    dinv_row = pl.pallas_call(
        _dinv_body,
        out_shape=jax.ShapeDtypeStruct((1, n_pad), jnp.float32),
    )(hist)
    dinv = dinv_row.reshape(n_pad, 1)[:n]

    mm = pl.pallas_call(
        _mm_body,
        out_shape=jax.ShapeDtypeStruct((n, d), jnp.float32),
    )
    epilogue = pl.pallas_call(
        _epilogue_body,
        out_shape=jax.ShapeDtypeStruct((n, d), jnp.float32),
    )

    h = x
    for (W, b, g, bt) in ((W1, b1, g1, bt1), (W2, b2, g2, bt2),
                          (W3, b3, g3, bt3)):
        hp = mm(h, W, dinv)
        parts = agg_kernel(hp, src2d, dst2d, zeros_rows)
        h = epilogue(parts[0, :n], parts[1, :n], hp, dinv,
                     b.reshape(1, d), g.reshape(1, d), bt.reshape(1, d))
    return h


# speedup vs baseline: 8.6612x; 1.0424x over previous
import functools

import jax
import jax.numpy as jnp
from jax import lax
from jax.experimental import pallas as pl
from jax.experimental.pallas import tpu as pltpu
from jax.experimental.pallas import tpu_sc as plsc

NC = 2
NS = 16
NW = NC * NS
L = 16

CHUNK = 128
NBUF = 2
BLKC = 32


def _pad_geometry(n, e):
    chunks_t = -(-e // (NW * CHUNK))
    chunks_t = -(-chunks_t // 8) * 8
    e_pad = NW * chunks_t * CHUNK
    rows_t = -(-(n + 1) // NS)
    rows_t = -(-rows_t // 8) * 8
    n_pad = rows_t * NS
    return chunks_t, e_pad, rows_t, n_pad


def _make_deg_kernel(n_pad, chunks_t):
    mesh = plsc.VectorSubcoreMesh(core_axis_name="c", subcore_axis_name="s")

    ept = chunks_t * CHUNK

    @functools.partial(
        pl.kernel,
        out_type=jax.ShapeDtypeStruct((NW, n_pad), jnp.float32),
        mesh=mesh,
        scratch_types=[
            pltpu.VMEM((ept,), jnp.int32),
            pltpu.VMEM((n_pad,), jnp.float32),
        ],
        compiler_params=pltpu.CompilerParams(needs_layout_passes=False),
    )
    def deg_kernel(dst_hbm, out_hbm, dst_v, hist_v):
        cid = lax.axis_index("c")
        sid = lax.axis_index("s")
        wid = cid * NS + sid
        pltpu.sync_copy(dst_hbm.at[pl.ds(wid * ept, ept)], dst_v)

        zeros16 = jnp.zeros((L,), jnp.float32)

        def zbody(i, carry):
            hist_v[pl.ds(i * L, L)] = zeros16
            return carry

        lax.fori_loop(0, n_pad // L, zbody, None)

        ones16 = jnp.ones((L,), jnp.float32)

        def body(i, carry):
            idx = dst_v[pl.ds(i * L, L)]
            plsc.addupdate_scatter(hist_v, [idx], ones16)
            return carry

        lax.fori_loop(0, ept // L, body, None)
        pltpu.sync_copy(hist_v, out_hbm.at[wid])

    return deg_kernel


def _make_agg_kernel(n, n_pad, rows_t, chunks_t):
    mesh = plsc.VectorSubcoreMesh(core_axis_name="c", subcore_axis_name="s")

    total_pair = 2 * chunks_t
    cpt0 = (total_pair * 4 // 5) // BLKC * BLKC
    cpt1 = total_pair - cpt0
    groups = BLKC // NBUF

    @functools.partial(
        pl.kernel,
        out_type=jax.ShapeDtypeStruct((NC, n_pad, 128), jnp.float32),
        mesh=mesh,
        scratch_types=[
            pltpu.VMEM((BLKC, CHUNK), jnp.int32),
            pltpu.VMEM((BLKC, CHUNK), jnp.int32),
            pltpu.VMEM((NBUF, CHUNK, 128), jnp.float32),
            pltpu.VMEM_SHARED((n_pad, 128), jnp.float32),
            pltpu.SemaphoreType.DMA((NBUF,)),
        ],
        compiler_params=pltpu.CompilerParams(needs_layout_passes=False),
    )
    def agg_kernel(hp_hbm, src_hbm, dst_hbm, zeros_hbm, out_hbm,
                   src_v, dst_v, rows_v, accum, sems):
        cid = lax.axis_index("c")
        sid = lax.axis_index("s")
        pltpu.sync_copy(zeros_hbm, accum.at[pl.ds(sid * rows_t, rows_t)])
        plsc.subcore_barrier()

        nblk = lax.select(cid == 0, cpt0 // BLKC, cpt1 // BLKC)
        base_chunk = lax.select(cid == 0, sid * cpt0,
                                NS * cpt0 + sid * cpt1)

        def blk_body(blk, carry):
            b0 = base_chunk + blk * BLKC
            pltpu.sync_copy(src_hbm.at[pl.ds(b0, BLKC)], src_v)
            pltpu.sync_copy(dst_hbm.at[pl.ds(b0, BLKC)], dst_v)

            for k in range(NBUF):
                pltpu.async_copy(hp_hbm.at[src_v.at[k]], rows_v.at[k],
                                 sems.at[k])

            def body(g, c2):
                for k in range(NBUF):
                    i = g * NBUF + k
                    pltpu.make_async_copy(
                        hp_hbm.at[src_v.at[i]], rows_v.at[k],
                        sems.at[k]).wait()
                    pltpu.sync_copy(rows_v.at[k], accum.at[dst_v.at[i]],
                                    add=True)

                    @pl.when(g + 1 < groups)
                    def _():
                        pltpu.async_copy(hp_hbm.at[src_v.at[i + NBUF]],
                                         rows_v.at[k], sems.at[k])

                return c2

            lax.fori_loop(0, groups, body, None)
            return carry

        lax.fori_loop(0, nblk, blk_body, None)

        plsc.subcore_barrier()
        pltpu.sync_copy(accum.at[pl.ds(sid * rows_t, rows_t)],
                        out_hbm.at[cid, pl.ds(sid * rows_t, rows_t)])

    return agg_kernel


def _dinv_body(hist_ref, o_ref):
    deg = jnp.sum(hist_ref[...], axis=0, keepdims=True) + 1.0
    o_ref[...] = lax.rsqrt(deg)


def _mm_body(x_ref, w_ref, dinv_ref, o_ref):
    h = jnp.dot(x_ref[...], w_ref[...], preferred_element_type=jnp.float32)
    o_ref[...] = h * dinv_ref[...]


def _epilogue_body(p0_ref, p1_ref, hp_ref, dinv_ref, b_ref, g_ref, bt_ref,
                   o_ref):
    t = (p0_ref[...] + p1_ref[...] + hp_ref[...]) * dinv_ref[...] + b_ref[...]
    mu = jnp.mean(t, axis=0, keepdims=True)
    var = jnp.mean((t - mu) ** 2, axis=0, keepdims=True)
    y = g_ref[...] * (t - mu) * lax.rsqrt(var + 1e-5) + bt_ref[...]
    o_ref[...] = jnp.maximum(y, 0.0)


def kernel(x, edge_index, W1, b1, g1, bt1, W2, b2, g2, bt2, W3, b3, g3, bt3):
    n, d = x.shape
    e = edge_index.shape[1]
    chunks_t, e_pad, rows_t, n_pad = _pad_geometry(n, e)

    src = edge_index[0]
    dst = edge_index[1]
    pad = e_pad - e
    src_p = jnp.concatenate([src, jnp.zeros((pad,), src.dtype)])
    dst_p = jnp.concatenate([dst, jnp.full((pad,), n, dst.dtype)])
    src2d = src_p.reshape(NW * chunks_t, CHUNK)
    dst2d = dst_p.reshape(NW * chunks_t, CHUNK)
    zeros_rows = jnp.zeros((rows_t, d), jnp.float32)

    deg_kernel = _make_deg_kernel(n_pad, chunks_t)
    agg_kernel = _make_agg_kernel(n, n_pad, rows_t, chunks_t)

    hist = deg_kernel(dst_p)

    dinv_row = pl.pallas_call(
        _dinv_body,
        out_shape=jax.ShapeDtypeStruct((1, n_pad), jnp.float32),
    )(hist)
    dinv = dinv_row.reshape(n_pad, 1)[:n]

    mm = pl.pallas_call(
        _mm_body,
        out_shape=jax.ShapeDtypeStruct((n, d), jnp.float32),
    )
    epilogue = pl.pallas_call(
        _epilogue_body,
        out_shape=jax.ShapeDtypeStruct((n, d), jnp.float32),
    )

    h = x
    for (W, b, g, bt) in ((W1, b1, g1, bt1), (W2, b2, g2, bt2),
                          (W3, b3, g3, bt3)):
        hp = mm(h, W, dinv)
        parts = agg_kernel(hp, src2d, dst2d, zeros_rows)
        h = epilogue(parts[0, :n], parts[1, :n], hp, dinv,
                     b.reshape(1, d), g.reshape(1, d), bt.reshape(1, d))
    return h
